# baseline (speedup 1.0000x reference)
import functools

import jax
import jax.numpy as jnp
from jax import lax
from jax.experimental import pallas as pl
from jax.experimental.pallas import tpu as pltpu
from jax.experimental.pallas import tpu_sc as plsc

N = 10000
E = 320000
D = 128
CLS_DIM = 128
NUM_CLS = 1000

NC = 2
NS = 16
NW = NC * NS
EPW = E // NW
ECHUNK = 128
EROWS = 2560
RPW = EROWS // NW
HCHUNK = 2000
NPAD = 10240
_HIST_G = 1280
RPTP = NPAD // NS

_sc_mesh = plsc.VectorSubcoreMesh(
    core_axis_name="c", subcore_axis_name="s", num_cores=NC, num_subcores=NS)


@functools.partial(
    pl.kernel,
    out_type=jax.ShapeDtypeStruct((NPAD // _HIST_G, NW, _HIST_G), jnp.float32),
    mesh=_sc_mesh,
    scratch_types=[
        pltpu.VMEM((NPAD,), jnp.float32),
        pltpu.VMEM((HCHUNK,), jnp.int32),
    ],
    compiler_params=pltpu.CompilerParams(needs_layout_passes=False),
)
def _hist_kernel(dst_hbm, hist_out, hist_v, idx_v):
    c = lax.axis_index("c")
    s = lax.axis_index("s")
    wid = c * NS + s

    def zbody(j, _):
        hist_v[pl.ds(j * 16, 16)] = jnp.zeros((16,), jnp.float32)
        return 0

    lax.fori_loop(0, NPAD // 16, zbody, 0)

    ones = jnp.ones((16,), jnp.float32)

    def chunk_body(i, _):
        pltpu.sync_copy(dst_hbm.at[pl.ds(wid * EPW + i * HCHUNK, HCHUNK)], idx_v)

        def inner(j, _):
            idx = idx_v[pl.ds(j * 16, 16)]
            plsc.addupdate_scatter(hist_v, [idx], ones)
            return 0

        lax.fori_loop(0, HCHUNK // 16, inner, 0)
        return 0

    lax.fori_loop(0, EPW // HCHUNK, chunk_body, 0)
    for g in range(NPAD // _HIST_G):
        pltpu.sync_copy(hist_v.at[pl.ds(g * _HIST_G, _HIST_G)],
                        hist_out.at[g, wid])


_ATAN_COEF = (
    0.9999999997078624, -0.33333325677909525, 0.19999666967905463,
    -0.14280017502794023, 0.11060457389637045, -0.08822392769760169,
    0.06772865569989875, -0.04519816824222727, 0.02302416556181928,
    -0.007568498443339108, 0.0011681262903039157,
)


def _prep_body(xe_ref, xh_ref, xs_ref, w_ref, hist_ref, hs_ref, dinv_ref):
    xe = xe_ref[...]
    xh = xh_ref[...]
    xs = xs_ref[...]

    nh = jnp.sqrt(jnp.sum(xh * xh, axis=1, keepdims=True))
    nhc = jnp.clip(nh, 1e-8, 1.0 - 1e-5)
    ath = 0.5 * jnp.log((1.0 + nhc) / (1.0 - nhc))
    xh = ath * xh / jnp.maximum(nh, 1e-8)

    nsr = jnp.sqrt(jnp.sum(xs * xs, axis=1, keepdims=True))
    nsc = jnp.maximum(nsr, 1e-8)
    t = jnp.minimum(nsc, 1.0 / nsc)
    u = t * t
    p = _ATAN_COEF[-1]
    for coef in _ATAN_COEF[-2::-1]:
        p = p * u + coef
    p = p * t
    atn = jnp.where(nsc <= 1.0, p, (jnp.pi / 2) - p)
    xs = atn * xs / nsc

    w = w_ref[...]
    h = jnp.dot(xe, w[0:D], preferred_element_type=jnp.float32)
    h += jnp.dot(xh, w[D:2 * D], preferred_element_type=jnp.float32)
    h += jnp.dot(xs, w[2 * D:3 * D], preferred_element_type=jnp.float32)

    deg = jnp.sum(hist_ref[0], axis=0, keepdims=True) + 1.0
    dinv = lax.rsqrt(deg)
    hs_ref[...] = h * dinv.T
    dinv_ref[...] = dinv.T


_PREP_R = 1280


def _tc_prep(x_E, x_H, x_S, W, hist):
    return pl.pallas_call(
        _prep_body,
        grid=(pl.cdiv(N, _PREP_R),),
        in_specs=[
            pl.BlockSpec((_PREP_R, D), lambda i: (i, 0)),
            pl.BlockSpec((_PREP_R, D), lambda i: (i, 0)),
            pl.BlockSpec((_PREP_R, D), lambda i: (i, 0)),
            pl.BlockSpec((3 * D, CLS_DIM), lambda i: (0, 0)),
            pl.BlockSpec((1, NW, _HIST_G), lambda i: (i, 0, 0)),
        ],
        out_specs=[
            pl.BlockSpec((_PREP_R, CLS_DIM), lambda i: (i, 0)),
            pl.BlockSpec((_PREP_R, 1), lambda i: (i, 0)),
        ],
        out_shape=[
            jax.ShapeDtypeStruct((NPAD, CLS_DIM), jnp.float32),
            jax.ShapeDtypeStruct((N, 1), jnp.float32),
        ],
    )(x_E, x_H, x_S, W, hist)


FAST_CORE = 0
IBATCH = 8
ROWS_FAST = 152
ROWS_SLOW = 8


@functools.partial(
    pl.kernel,
    out_type=jax.ShapeDtypeStruct((NC, NPAD, CLS_DIM), jnp.float32),
    mesh=_sc_mesh,
    scratch_types=[
        pltpu.VMEM_SHARED((NPAD, CLS_DIM), jnp.float32),
        pltpu.VMEM((8, ECHUNK), jnp.int32),
        pltpu.VMEM((8, ECHUNK), jnp.int32),
        pltpu.VMEM((2, ECHUNK, CLS_DIM), jnp.float32),
        pltpu.SemaphoreType.DMA,
        pltpu.SemaphoreType.DMA,
    ],
)
def _scatter_kernel(hs_hbm, src_hbm, dst_hbm, s_out,
                    acc, srcb, dstb, rows, gsem, ssem):
    c = lax.axis_index("c")
    s = lax.axis_index("s")
    rbase = s * RPTP

    pltpu.sync_copy(hs_hbm.at[pl.ds(rbase, RPTP)], acc.at[pl.ds(rbase, RPTP)])
    plsc.subcore_barrier()

    def run(nbatch, tile_base):
        def batch(bi, _):
            base = tile_base + bi * IBATCH
            pltpu.sync_copy(src_hbm.at[pl.ds(base, IBATCH)], srcb)
            pltpu.sync_copy(dst_hbm.at[pl.ds(base, IBATCH)], dstb)
            g = {}
            sc = {}
            g[0] = pltpu.async_copy(hs_hbm.at[srcb.at[0]], rows.at[0], gsem)
            for j in range(IBATCH):
                if 1 <= j and j + 1 < IBATCH:
                    sc[j - 1].wait()
                if j + 1 < IBATCH:
                    g[j + 1] = pltpu.async_copy(
                        hs_hbm.at[srcb.at[j + 1]], rows.at[(j + 1) % 2], gsem)
                g[j].wait()
                sc[j] = pltpu.async_copy(
                    rows.at[j % 2], acc.at[dstb.at[j]], ssem, add=True)
            sc[IBATCH - 2].wait()
            sc[IBATCH - 1].wait()
            return 0

        lax.fori_loop(0, nbatch, batch, 0)

    @pl.when(c == FAST_CORE)
    def _():
        run(ROWS_FAST // IBATCH, s * ROWS_FAST)

    @pl.when(c != FAST_CORE)
    def _():
        run(ROWS_SLOW // IBATCH, NS * ROWS_FAST + s * ROWS_SLOW)

    plsc.subcore_barrier()
    pltpu.sync_copy(acc.at[pl.ds(rbase, RPTP)], s_out.at[c, pl.ds(rbase, RPTP)])


def _out_body(s_ref, hs_ref, dinv_ref, b_ref, cls_ref, out_ref):
    y = (s_ref[0] + s_ref[1] - hs_ref[...]) * dinv_ref[...] + b_ref[...]
    yn = jnp.sqrt(jnp.sum(y * y, axis=1, keepdims=True))
    y = y / jnp.maximum(yn, 1e-8)
    cemb = cls_ref[...]
    cn = jnp.sqrt(jnp.sum(cemb * cemb, axis=1, keepdims=True))
    cemb = cemb / jnp.maximum(cn, 1e-8)
    out_ref[...] = lax.dot_general(
        y, cemb, (((1,), (1,)), ((), ())),
        preferred_element_type=jnp.float32)


_OUT_R = 1000


def _tc_out(s_part, hs, dinv, b2, cls_embeddings):
    return pl.pallas_call(
        _out_body,
        grid=(N // _OUT_R,),
        in_specs=[
            pl.BlockSpec((NC, _OUT_R, CLS_DIM), lambda i: (0, i, 0)),
            pl.BlockSpec((_OUT_R, CLS_DIM), lambda i: (i, 0)),
            pl.BlockSpec((_OUT_R, 1), lambda i: (i, 0)),
            pl.BlockSpec((1, CLS_DIM), lambda i: (0, 0)),
            pl.BlockSpec((NUM_CLS, CLS_DIM), lambda i: (0, 0)),
        ],
        out_specs=pl.BlockSpec((_OUT_R, NUM_CLS), lambda i: (i, 0)),
        out_shape=jax.ShapeDtypeStruct((N, NUM_CLS), jnp.float32),
    )(s_part, hs, dinv, b2, cls_embeddings)


def kernel(x_E, x_H, x_S, edge_index, W, b, cls_embeddings):
    npad = EROWS * ECHUNK - E
    ei = jnp.concatenate(
        [edge_index, jnp.full((2, npad), N, dtype=jnp.int32)], axis=1)
    src2 = ei[0].reshape(EROWS, ECHUNK)
    dst2 = ei[1].reshape(EROWS, ECHUNK)
    hist = _hist_kernel(edge_index[1])
    hs, dinv = _tc_prep(x_E, x_H, x_S, W, hist)
    s_part = _scatter_kernel(hs, src2, dst2)
    return _tc_out(s_part, hs, dinv, b.reshape(1, CLS_DIM), cls_embeddings)


# speedup vs baseline: 1.1345x; 1.1345x over previous
import functools

import jax
import jax.numpy as jnp
from jax import lax
from jax.experimental import pallas as pl
from jax.experimental.pallas import tpu as pltpu
from jax.experimental.pallas import tpu_sc as plsc

N = 10000
E = 320000
D = 128
CLS_DIM = 128
NUM_CLS = 1000

NC = 2
NS = 16
NW = NC * NS
EPW = E // NW
ECHUNK = 128
EROWS = 2560
RPW = EROWS // NW
HCHUNK = 2000
NPAD = 10240
_HIST_G = 1280
RPTP = NPAD // NS

_sc_mesh = plsc.VectorSubcoreMesh(
    core_axis_name="c", subcore_axis_name="s", num_cores=NC, num_subcores=NS)


@functools.partial(
    pl.kernel,
    out_type=jax.ShapeDtypeStruct((NPAD // _HIST_G, NW, _HIST_G), jnp.float32),
    mesh=_sc_mesh,
    scratch_types=[
        pltpu.VMEM((NPAD,), jnp.float32),
        pltpu.VMEM((HCHUNK,), jnp.int32),
    ],
    compiler_params=pltpu.CompilerParams(needs_layout_passes=False),
)
def _hist_kernel(dst_hbm, hist_out, hist_v, idx_v):
    c = lax.axis_index("c")
    s = lax.axis_index("s")
    wid = c * NS + s

    def zbody(j, _):
        hist_v[pl.ds(j * 16, 16)] = jnp.zeros((16,), jnp.float32)
        return 0

    lax.fori_loop(0, NPAD // 16, zbody, 0)

    ones = jnp.ones((16,), jnp.float32)

    def chunk_body(i, _):
        pltpu.sync_copy(dst_hbm.at[pl.ds(wid * EPW + i * HCHUNK, HCHUNK)], idx_v)

        def inner(j, _):
            idx = idx_v[pl.ds(j * 16, 16)]
            plsc.addupdate_scatter(hist_v, [idx], ones)
            return 0

        lax.fori_loop(0, HCHUNK // 16, inner, 0)
        return 0

    lax.fori_loop(0, EPW // HCHUNK, chunk_body, 0)
    for g in range(NPAD // _HIST_G):
        pltpu.sync_copy(hist_v.at[pl.ds(g * _HIST_G, _HIST_G)],
                        hist_out.at[g, wid])


_ATAN_COEF = (
    0.9999999997078624, -0.33333325677909525, 0.19999666967905463,
    -0.14280017502794023, 0.11060457389637045, -0.08822392769760169,
    0.06772865569989875, -0.04519816824222727, 0.02302416556181928,
    -0.007568498443339108, 0.0011681262903039157,
)


def _prep_body(xe_ref, xh_ref, xs_ref, w_ref, hist_ref, hs_ref, dinv_ref):
    xe = xe_ref[...]
    xh = xh_ref[...]
    xs = xs_ref[...]

    nh = jnp.sqrt(jnp.sum(xh * xh, axis=1, keepdims=True))
    nhc = jnp.clip(nh, 1e-8, 1.0 - 1e-5)
    ath = 0.5 * jnp.log((1.0 + nhc) / (1.0 - nhc))
    xh = ath * xh / jnp.maximum(nh, 1e-8)

    nsr = jnp.sqrt(jnp.sum(xs * xs, axis=1, keepdims=True))
    nsc = jnp.maximum(nsr, 1e-8)
    t = jnp.minimum(nsc, 1.0 / nsc)
    u = t * t
    p = _ATAN_COEF[-1]
    for coef in _ATAN_COEF[-2::-1]:
        p = p * u + coef
    p = p * t
    atn = jnp.where(nsc <= 1.0, p, (jnp.pi / 2) - p)
    xs = atn * xs / nsc

    w = w_ref[...]
    h = jnp.dot(xe, w[0:D], preferred_element_type=jnp.float32)
    h += jnp.dot(xh, w[D:2 * D], preferred_element_type=jnp.float32)
    h += jnp.dot(xs, w[2 * D:3 * D], preferred_element_type=jnp.float32)

    deg = jnp.sum(hist_ref[0], axis=0, keepdims=True) + 1.0
    dinv = lax.rsqrt(deg)
    hs_ref[...] = h * dinv.T
    dinv_ref[...] = dinv.T


_PREP_R = 1280


def _tc_prep(x_E, x_H, x_S, W, hist):
    return pl.pallas_call(
        _prep_body,
        grid=(pl.cdiv(N, _PREP_R),),
        in_specs=[
            pl.BlockSpec((_PREP_R, D), lambda i: (i, 0)),
            pl.BlockSpec((_PREP_R, D), lambda i: (i, 0)),
            pl.BlockSpec((_PREP_R, D), lambda i: (i, 0)),
            pl.BlockSpec((3 * D, CLS_DIM), lambda i: (0, 0)),
            pl.BlockSpec((1, NW, _HIST_G), lambda i: (i, 0, 0)),
        ],
        out_specs=[
            pl.BlockSpec((_PREP_R, CLS_DIM), lambda i: (i, 0)),
            pl.BlockSpec((_PREP_R, 1), lambda i: (i, 0)),
        ],
        out_shape=[
            jax.ShapeDtypeStruct((NPAD, CLS_DIM), jnp.float32),
            jax.ShapeDtypeStruct((N, 1), jnp.float32),
        ],
    )(x_E, x_H, x_S, W, hist)


CHUNK = 80


@functools.partial(
    pl.kernel,
    out_type=jax.ShapeDtypeStruct((NC, NPAD, CLS_DIM), jnp.float32),
    mesh=_sc_mesh,
    scratch_types=[
        pltpu.VMEM_SHARED((NPAD, CLS_DIM), jnp.float32),
        pltpu.VMEM((CHUNK,), jnp.int32),
        pltpu.VMEM((CHUNK,), jnp.int32),
        pltpu.VMEM((CHUNK, CLS_DIM), jnp.float32),
        pltpu.SemaphoreType.DMA,
    ],
)
def _scatter_kernel(hs_hbm, src_hbm, dst_hbm, s_out,
                    acc, src_v, dst_v, rows_v, gsem):
    c = lax.axis_index("c")
    s = lax.axis_index("s")
    wid = c * NS + s
    rbase = s * RPTP

    pltpu.sync_copy(hs_hbm.at[pl.ds(rbase, RPTP)], acc.at[pl.ds(rbase, RPTP)])
    plsc.subcore_barrier()

    ebase = wid * EPW

    def body(i, _):
        eb = ebase + i * CHUNK
        pltpu.sync_copy(src_hbm.at[pl.ds(eb, CHUNK)], src_v)
        pltpu.sync_copy(dst_hbm.at[pl.ds(eb, CHUNK)], dst_v)
        pltpu.async_copy(hs_hbm.at[src_v], rows_v, gsem).wait()
        pltpu.sync_copy(rows_v, acc.at[dst_v], add=True)
        return 0

    lax.fori_loop(0, EPW // CHUNK, body, 0)
    plsc.subcore_barrier()
    pltpu.sync_copy(acc.at[pl.ds(rbase, RPTP)], s_out.at[c, pl.ds(rbase, RPTP)])


def _out_body(s_ref, hs_ref, dinv_ref, b_ref, cls_ref, out_ref):
    y = (s_ref[0] + s_ref[1] - hs_ref[...]) * dinv_ref[...] + b_ref[...]
    yn = jnp.sqrt(jnp.sum(y * y, axis=1, keepdims=True))
    y = y / jnp.maximum(yn, 1e-8)
    cemb = cls_ref[...]
    cn = jnp.sqrt(jnp.sum(cemb * cemb, axis=1, keepdims=True))
    cemb = cemb / jnp.maximum(cn, 1e-8)
    out_ref[...] = lax.dot_general(
        y, cemb, (((1,), (1,)), ((), ())),
        preferred_element_type=jnp.float32)


_OUT_R = 1000


def _tc_out(s_part, hs, dinv, b2, cls_embeddings):
    return pl.pallas_call(
        _out_body,
        grid=(N // _OUT_R,),
        in_specs=[
            pl.BlockSpec((NC, _OUT_R, CLS_DIM), lambda i: (0, i, 0)),
            pl.BlockSpec((_OUT_R, CLS_DIM), lambda i: (i, 0)),
            pl.BlockSpec((_OUT_R, 1), lambda i: (i, 0)),
            pl.BlockSpec((1, CLS_DIM), lambda i: (0, 0)),
            pl.BlockSpec((NUM_CLS, CLS_DIM), lambda i: (0, 0)),
        ],
        out_specs=pl.BlockSpec((_OUT_R, NUM_CLS), lambda i: (i, 0)),
        out_shape=jax.ShapeDtypeStruct((N, NUM_CLS), jnp.float32),
    )(s_part, hs, dinv, b2, cls_embeddings)


def kernel(x_E, x_H, x_S, edge_index, W, b, cls_embeddings):
    hist = _hist_kernel(edge_index[1])
    hs, dinv = _tc_prep(x_E, x_H, x_S, W, hist)
    s_part = _scatter_kernel(hs, edge_index[0], edge_index[1])
    return _tc_out(s_part, hs, dinv, b.reshape(1, CLS_DIM), cls_embeddings)


# speedup vs baseline: 1.4592x; 1.2862x over previous
import functools

import jax
import jax.numpy as jnp
from jax import lax
from jax.experimental import pallas as pl
from jax.experimental.pallas import tpu as pltpu
from jax.experimental.pallas import tpu_sc as plsc

N = 10000
E = 320000
D = 128
CLS_DIM = 128
NUM_CLS = 1000

NC = 2
NS = 16
NW = NC * NS
EPW = E // NW
ECHUNK = 128
EROWS = 2560
RPW = EROWS // NW
HCHUNK = 2000
NPAD = 10240
_HIST_G = 1280
RPTP = NPAD // NS

_sc_mesh = plsc.VectorSubcoreMesh(
    core_axis_name="c", subcore_axis_name="s", num_cores=NC, num_subcores=NS)


@functools.partial(
    pl.kernel,
    out_type=jax.ShapeDtypeStruct((NPAD // _HIST_G, NW, _HIST_G), jnp.float32),
    mesh=_sc_mesh,
    scratch_types=[
        pltpu.VMEM((NPAD,), jnp.float32),
        pltpu.VMEM((HCHUNK,), jnp.int32),
    ],
    compiler_params=pltpu.CompilerParams(needs_layout_passes=False),
)
def _hist_kernel(dst_hbm, hist_out, hist_v, idx_v):
    c = lax.axis_index("c")
    s = lax.axis_index("s")
    wid = c * NS + s

    def zbody(j, _):
        hist_v[pl.ds(j * 16, 16)] = jnp.zeros((16,), jnp.float32)
        return 0

    lax.fori_loop(0, NPAD // 16, zbody, 0)

    ones = jnp.ones((16,), jnp.float32)

    def chunk_body(i, _):
        pltpu.sync_copy(dst_hbm.at[pl.ds(wid * EPW + i * HCHUNK, HCHUNK)], idx_v)

        def inner(j, _):
            idx = idx_v[pl.ds(j * 16, 16)]
            plsc.addupdate_scatter(hist_v, [idx], ones)
            return 0

        lax.fori_loop(0, HCHUNK // 16, inner, 0)
        return 0

    lax.fori_loop(0, EPW // HCHUNK, chunk_body, 0)
    for g in range(NPAD // _HIST_G):
        pltpu.sync_copy(hist_v.at[pl.ds(g * _HIST_G, _HIST_G)],
                        hist_out.at[g, wid])


_ATAN_COEF = (
    0.9999999997078624, -0.33333325677909525, 0.19999666967905463,
    -0.14280017502794023, 0.11060457389637045, -0.08822392769760169,
    0.06772865569989875, -0.04519816824222727, 0.02302416556181928,
    -0.007568498443339108, 0.0011681262903039157,
)


def _prep_body(xe_ref, xh_ref, xs_ref, w_ref, hist_ref, hs_ref, dinv_ref):
    xe = xe_ref[...]
    xh = xh_ref[...]
    xs = xs_ref[...]

    nh = jnp.sqrt(jnp.sum(xh * xh, axis=1, keepdims=True))
    nhc = jnp.clip(nh, 1e-8, 1.0 - 1e-5)
    ath = 0.5 * jnp.log((1.0 + nhc) / (1.0 - nhc))
    xh = ath * xh / jnp.maximum(nh, 1e-8)

    nsr = jnp.sqrt(jnp.sum(xs * xs, axis=1, keepdims=True))
    nsc = jnp.maximum(nsr, 1e-8)
    t = jnp.minimum(nsc, 1.0 / nsc)
    u = t * t
    p = _ATAN_COEF[-1]
    for coef in _ATAN_COEF[-2::-1]:
        p = p * u + coef
    p = p * t
    atn = jnp.where(nsc <= 1.0, p, (jnp.pi / 2) - p)
    xs = atn * xs / nsc

    w = w_ref[...]
    h = jnp.dot(xe, w[0:D], preferred_element_type=jnp.float32)
    h += jnp.dot(xh, w[D:2 * D], preferred_element_type=jnp.float32)
    h += jnp.dot(xs, w[2 * D:3 * D], preferred_element_type=jnp.float32)

    deg = jnp.sum(hist_ref[0], axis=0, keepdims=True) + 1.0
    dinv = lax.rsqrt(deg)
    hs_ref[...] = h * dinv.T
    dinv_ref[...] = dinv.T


_PREP_R = 1280


def _tc_prep(x_E, x_H, x_S, W, hist):
    return pl.pallas_call(
        _prep_body,
        grid=(pl.cdiv(N, _PREP_R),),
        in_specs=[
            pl.BlockSpec((_PREP_R, D), lambda i: (i, 0)),
            pl.BlockSpec((_PREP_R, D), lambda i: (i, 0)),
            pl.BlockSpec((_PREP_R, D), lambda i: (i, 0)),
            pl.BlockSpec((3 * D, CLS_DIM), lambda i: (0, 0)),
            pl.BlockSpec((1, NW, _HIST_G), lambda i: (i, 0, 0)),
        ],
        out_specs=[
            pl.BlockSpec((_PREP_R, CLS_DIM), lambda i: (i, 0)),
            pl.BlockSpec((_PREP_R, 1), lambda i: (i, 0)),
        ],
        out_shape=[
            jax.ShapeDtypeStruct((NPAD, CLS_DIM), jnp.float32),
            jax.ShapeDtypeStruct((N, 1), jnp.float32),
        ],
    )(x_E, x_H, x_S, W, hist)


CHUNK = 80
BIGC = 2000


@functools.partial(
    pl.kernel,
    out_type=jax.ShapeDtypeStruct((NC, NPAD, CLS_DIM), jnp.float32),
    mesh=_sc_mesh,
    scratch_types=[
        pltpu.VMEM_SHARED((NPAD, CLS_DIM), jnp.float32),
        pltpu.VMEM((BIGC,), jnp.int32),
        pltpu.VMEM((BIGC,), jnp.int32),
        pltpu.VMEM((CHUNK,), jnp.int32),
        pltpu.VMEM((CHUNK, CLS_DIM), jnp.float32),
        pltpu.SemaphoreType.DMA,
    ],
)
def _scatter_kernel(hs_hbm, src_hbm, dst_hbm, s_out,
                    acc, srcbig, dstbig, dst_v, rows_v, gsem):
    c = lax.axis_index("c")
    s = lax.axis_index("s")
    wid = c * NS + s
    rbase = s * RPTP

    pltpu.sync_copy(hs_hbm.at[pl.ds(rbase, RPTP)], acc.at[pl.ds(rbase, RPTP)])
    plsc.subcore_barrier()

    ebase = wid * EPW

    def big_body(bi, _):
        bb = ebase + bi * BIGC
        pltpu.sync_copy(src_hbm.at[pl.ds(bb, BIGC)], srcbig)
        pltpu.sync_copy(dst_hbm.at[pl.ds(bb, BIGC)], dstbig)

        def chunk(k, _):
            pltpu.async_copy(
                hs_hbm.at[srcbig.at[pl.ds(k * CHUNK, CHUNK)]],
                rows_v, gsem).wait()
            for v in range(CHUNK // 16):
                dst_v[pl.ds(v * 16, 16)] = dstbig[pl.ds(k * CHUNK + v * 16, 16)]
            pltpu.sync_copy(rows_v, acc.at[dst_v], add=True)
            return 0

        lax.fori_loop(0, BIGC // CHUNK, chunk, 0)
        return 0

    lax.fori_loop(0, EPW // BIGC, big_body, 0)
    plsc.subcore_barrier()
    pltpu.sync_copy(acc.at[pl.ds(rbase, RPTP)], s_out.at[c, pl.ds(rbase, RPTP)])


def _out_body(s_ref, hs_ref, dinv_ref, b_ref, cls_ref, out_ref):
    y = (s_ref[0] + s_ref[1] - hs_ref[...]) * dinv_ref[...] + b_ref[...]
    yn = jnp.sqrt(jnp.sum(y * y, axis=1, keepdims=True))
    y = y / jnp.maximum(yn, 1e-8)
    cemb = cls_ref[...]
    cn = jnp.sqrt(jnp.sum(cemb * cemb, axis=1, keepdims=True))
    cemb = cemb / jnp.maximum(cn, 1e-8)
    out_ref[...] = lax.dot_general(
        y, cemb, (((1,), (1,)), ((), ())),
        preferred_element_type=jnp.float32)


_OUT_R = 1000


def _tc_out(s_part, hs, dinv, b2, cls_embeddings):
    return pl.pallas_call(
        _out_body,
        grid=(N // _OUT_R,),
        in_specs=[
            pl.BlockSpec((NC, _OUT_R, CLS_DIM), lambda i: (0, i, 0)),
            pl.BlockSpec((_OUT_R, CLS_DIM), lambda i: (i, 0)),
            pl.BlockSpec((_OUT_R, 1), lambda i: (i, 0)),
            pl.BlockSpec((1, CLS_DIM), lambda i: (0, 0)),
            pl.BlockSpec((NUM_CLS, CLS_DIM), lambda i: (0, 0)),
        ],
        out_specs=pl.BlockSpec((_OUT_R, NUM_CLS), lambda i: (i, 0)),
        out_shape=jax.ShapeDtypeStruct((N, NUM_CLS), jnp.float32),
    )(s_part, hs, dinv, b2, cls_embeddings)


def kernel(x_E, x_H, x_S, edge_index, W, b, cls_embeddings):
    hist = _hist_kernel(edge_index[1])
    hs, dinv = _tc_prep(x_E, x_H, x_S, W, hist)
    s_part = _scatter_kernel(hs, edge_index[0], edge_index[1])
    return _tc_out(s_part, hs, dinv, b.reshape(1, CLS_DIM), cls_embeddings)


# speedup vs baseline: 1.9651x; 1.3467x over previous
import functools

import jax
import jax.numpy as jnp
from jax import lax
from jax.experimental import pallas as pl
from jax.experimental.pallas import tpu as pltpu
from jax.experimental.pallas import tpu_sc as plsc

N = 10000
E = 320000
D = 128
CLS_DIM = 128
NUM_CLS = 1000

NC = 2
NS = 16
NW = NC * NS
EPW = E // NW
ECHUNK = 128
EROWS = 2560
RPW = EROWS // NW
HCHUNK = 2000
NPAD = 10240
_HIST_G = 1280
RPTP = NPAD // NS

_sc_mesh = plsc.VectorSubcoreMesh(
    core_axis_name="c", subcore_axis_name="s", num_cores=NC, num_subcores=NS)


@functools.partial(
    pl.kernel,
    out_type=jax.ShapeDtypeStruct((NPAD // _HIST_G, NW, _HIST_G), jnp.float32),
    mesh=_sc_mesh,
    scratch_types=[
        pltpu.VMEM((NPAD,), jnp.float32),
        pltpu.VMEM((HCHUNK,), jnp.int32),
    ],
    compiler_params=pltpu.CompilerParams(needs_layout_passes=False),
)
def _hist_kernel(dst_hbm, hist_out, hist_v, idx_v):
    c = lax.axis_index("c")
    s = lax.axis_index("s")
    wid = c * NS + s

    def zbody(j, _):
        hist_v[pl.ds(j * 16, 16)] = jnp.zeros((16,), jnp.float32)
        return 0

    lax.fori_loop(0, NPAD // 16, zbody, 0)

    ones = jnp.ones((16,), jnp.float32)

    def chunk_body(i, _):
        pltpu.sync_copy(dst_hbm.at[pl.ds(wid * EPW + i * HCHUNK, HCHUNK)], idx_v)

        def inner(j, _):
            idx = idx_v[pl.ds(j * 16, 16)]
            plsc.addupdate_scatter(hist_v, [idx], ones)
            return 0

        lax.fori_loop(0, HCHUNK // 16, inner, 0)
        return 0

    lax.fori_loop(0, EPW // HCHUNK, chunk_body, 0)
    for g in range(NPAD // _HIST_G):
        pltpu.sync_copy(hist_v.at[pl.ds(g * _HIST_G, _HIST_G)],
                        hist_out.at[g, wid])


_ATAN_COEF = (
    0.9999999997078624, -0.33333325677909525, 0.19999666967905463,
    -0.14280017502794023, 0.11060457389637045, -0.08822392769760169,
    0.06772865569989875, -0.04519816824222727, 0.02302416556181928,
    -0.007568498443339108, 0.0011681262903039157,
)


def _prep_body(xe_ref, xh_ref, xs_ref, w_ref, hist_ref, hs_ref, dinv_ref):
    xe = xe_ref[...]
    xh = xh_ref[...]
    xs = xs_ref[...]

    nh = jnp.sqrt(jnp.sum(xh * xh, axis=1, keepdims=True))
    nhc = jnp.clip(nh, 1e-8, 1.0 - 1e-5)
    ath = 0.5 * jnp.log((1.0 + nhc) / (1.0 - nhc))
    xh = ath * xh / jnp.maximum(nh, 1e-8)

    nsr = jnp.sqrt(jnp.sum(xs * xs, axis=1, keepdims=True))
    nsc = jnp.maximum(nsr, 1e-8)
    t = jnp.minimum(nsc, 1.0 / nsc)
    u = t * t
    p = _ATAN_COEF[-1]
    for coef in _ATAN_COEF[-2::-1]:
        p = p * u + coef
    p = p * t
    atn = jnp.where(nsc <= 1.0, p, (jnp.pi / 2) - p)
    xs = atn * xs / nsc

    w = w_ref[...]
    h = jnp.dot(xe, w[0:D], preferred_element_type=jnp.float32)
    h += jnp.dot(xh, w[D:2 * D], preferred_element_type=jnp.float32)
    h += jnp.dot(xs, w[2 * D:3 * D], preferred_element_type=jnp.float32)

    deg = jnp.sum(hist_ref[0], axis=0, keepdims=True) + 1.0
    dinv = lax.rsqrt(deg)
    hs_ref[...] = h * dinv.T
    dinv_ref[...] = dinv.T


_PREP_R = 1280


def _tc_prep(x_E, x_H, x_S, W, hist):
    return pl.pallas_call(
        _prep_body,
        grid=(pl.cdiv(N, _PREP_R),),
        in_specs=[
            pl.BlockSpec((_PREP_R, D), lambda i: (i, 0)),
            pl.BlockSpec((_PREP_R, D), lambda i: (i, 0)),
            pl.BlockSpec((_PREP_R, D), lambda i: (i, 0)),
            pl.BlockSpec((3 * D, CLS_DIM), lambda i: (0, 0)),
            pl.BlockSpec((1, NW, _HIST_G), lambda i: (i, 0, 0)),
        ],
        out_specs=[
            pl.BlockSpec((_PREP_R, CLS_DIM), lambda i: (i, 0)),
            pl.BlockSpec((_PREP_R, 1), lambda i: (i, 0)),
        ],
        out_shape=[
            jax.ShapeDtypeStruct((NPAD, CLS_DIM), jnp.float32),
            jax.ShapeDtypeStruct((N, 1), jnp.float32),
        ],
    )(x_E, x_H, x_S, W, hist)


CHUNK = 80
BIGC = 2000


@functools.partial(
    pl.kernel,
    out_type=jax.ShapeDtypeStruct((NC, NPAD, CLS_DIM), jnp.float32),
    mesh=_sc_mesh,
    scratch_types=[
        pltpu.VMEM_SHARED((NPAD, CLS_DIM), jnp.float32),
        pltpu.VMEM((BIGC,), jnp.int32),
        pltpu.VMEM((BIGC,), jnp.int32),
        pltpu.VMEM((CHUNK,), jnp.int32),
        pltpu.VMEM((2, CHUNK, CLS_DIM), jnp.float32),
        pltpu.SemaphoreType.DMA,
    ],
)
def _scatter_kernel(hs_hbm, src_hbm, dst_hbm, s_out,
                    acc, srcbig, dstbig, dst_v, rows_v, gsem):
    c = lax.axis_index("c")
    s = lax.axis_index("s")
    wid = c * NS + s
    rbase = s * RPTP

    pltpu.sync_copy(hs_hbm.at[pl.ds(rbase, RPTP)], acc.at[pl.ds(rbase, RPTP)])
    plsc.subcore_barrier()

    ebase = wid * EPW

    def _drain(slot):
        pltpu.make_async_copy(hs_hbm.at[pl.ds(0, CHUNK)],
                              rows_v.at[slot], gsem).wait()

    def _scat(k, slot):
        for v in range(CHUNK // 16):
            dst_v[pl.ds(v * 16, 16)] = dstbig[pl.ds(k * CHUNK + v * 16, 16)]
        pltpu.sync_copy(rows_v.at[slot], acc.at[dst_v], add=True)

    def _gath(k, slot):
        pltpu.async_copy(hs_hbm.at[srcbig.at[pl.ds(k * CHUNK, CHUNK)]],
                         rows_v.at[slot], gsem)

    def big_body(bi, _):
        bb = ebase + bi * BIGC
        pltpu.sync_copy(src_hbm.at[pl.ds(bb, BIGC)], srcbig)
        pltpu.sync_copy(dst_hbm.at[pl.ds(bb, BIGC)], dstbig)
        _gath(0, 0)

        def pair(k2, _):
            k = 2 * k2
            _gath(k + 1, 1)
            _drain(0)
            _scat(k, 0)
            _gath(k + 2, 0)
            _drain(1)
            _scat(k + 1, 1)
            return 0

        lax.fori_loop(0, (BIGC // CHUNK) // 2, pair, 0)
        _drain(0)
        _scat(BIGC // CHUNK - 1, 0)
        return 0

    lax.fori_loop(0, EPW // BIGC, big_body, 0)
    plsc.subcore_barrier()
    pltpu.sync_copy(acc.at[pl.ds(rbase, RPTP)], s_out.at[c, pl.ds(rbase, RPTP)])


def _out_body(s_ref, hs_ref, dinv_ref, b_ref, cls_ref, out_ref):
    y = (s_ref[0] + s_ref[1] - hs_ref[...]) * dinv_ref[...] + b_ref[...]
    yn = jnp.sqrt(jnp.sum(y * y, axis=1, keepdims=True))
    y = y / jnp.maximum(yn, 1e-8)
    cemb = cls_ref[...]
    cn = jnp.sqrt(jnp.sum(cemb * cemb, axis=1, keepdims=True))
    cemb = cemb / jnp.maximum(cn, 1e-8)
    out_ref[...] = lax.dot_general(
        y, cemb, (((1,), (1,)), ((), ())),
        preferred_element_type=jnp.float32)


_OUT_R = 1000


def _tc_out(s_part, hs, dinv, b2, cls_embeddings):
    return pl.pallas_call(
        _out_body,
        grid=(N // _OUT_R,),
        in_specs=[
            pl.BlockSpec((NC, _OUT_R, CLS_DIM), lambda i: (0, i, 0)),
            pl.BlockSpec((_OUT_R, CLS_DIM), lambda i: (i, 0)),
            pl.BlockSpec((_OUT_R, 1), lambda i: (i, 0)),
            pl.BlockSpec((1, CLS_DIM), lambda i: (0, 0)),
            pl.BlockSpec((NUM_CLS, CLS_DIM), lambda i: (0, 0)),
        ],
        out_specs=pl.BlockSpec((_OUT_R, NUM_CLS), lambda i: (i, 0)),
        out_shape=jax.ShapeDtypeStruct((N, NUM_CLS), jnp.float32),
    )(s_part, hs, dinv, b2, cls_embeddings)


def kernel(x_E, x_H, x_S, edge_index, W, b, cls_embeddings):
    hist = _hist_kernel(edge_index[1])
    hs, dinv = _tc_prep(x_E, x_H, x_S, W, hist)
    s_part = _scatter_kernel(hs, edge_index[0], edge_index[1])
    return _tc_out(s_part, hs, dinv, b.reshape(1, CLS_DIM), cls_embeddings)


# speedup vs baseline: 2.2704x; 1.1553x over previous
import functools

import jax
import jax.numpy as jnp
from jax import lax
from jax.experimental import pallas as pl
from jax.experimental.pallas import tpu as pltpu
from jax.experimental.pallas import tpu_sc as plsc

N = 10000
E = 320000
D = 128
CLS_DIM = 128
NUM_CLS = 1000

NC = 2
NS = 16
NW = NC * NS
EPW = E // NW
ECHUNK = 128
EROWS = 2560
RPW = EROWS // NW
HCHUNK = 2000
NPAD = 10240
_HIST_G = 1280
RPTP = NPAD // NS

_sc_mesh = plsc.VectorSubcoreMesh(
    core_axis_name="c", subcore_axis_name="s", num_cores=NC, num_subcores=NS)


@functools.partial(
    pl.kernel,
    out_type=jax.ShapeDtypeStruct((NPAD // _HIST_G, NW, _HIST_G), jnp.float32),
    mesh=_sc_mesh,
    scratch_types=[
        pltpu.VMEM((NPAD,), jnp.float32),
        pltpu.VMEM((HCHUNK,), jnp.int32),
    ],
    compiler_params=pltpu.CompilerParams(needs_layout_passes=False),
)
def _hist_kernel(ei_hbm, hist_out, hist_v, idx_v):
    c = lax.axis_index("c")
    s = lax.axis_index("s")
    wid = c * NS + s

    def zbody(j, _):
        hist_v[pl.ds(j * 16, 16)] = jnp.zeros((16,), jnp.float32)
        return 0

    lax.fori_loop(0, NPAD // 16, zbody, 0)

    ones = jnp.ones((16,), jnp.float32)

    def chunk_body(i, _):
        pltpu.sync_copy(ei_hbm.at[pl.ds(E + wid * EPW + i * HCHUNK, HCHUNK)], idx_v)

        def inner(j, _):
            idx = idx_v[pl.ds(j * 16, 16)]
            plsc.addupdate_scatter(hist_v, [idx], ones)
            return 0

        lax.fori_loop(0, HCHUNK // 16, inner, 0)
        return 0

    lax.fori_loop(0, EPW // HCHUNK, chunk_body, 0)
    for g in range(NPAD // _HIST_G):
        pltpu.sync_copy(hist_v.at[pl.ds(g * _HIST_G, _HIST_G)],
                        hist_out.at[g, wid])


_ATAN_COEF = (
    0.9999999997078624, -0.33333325677909525, 0.19999666967905463,
    -0.14280017502794023, 0.11060457389637045, -0.08822392769760169,
    0.06772865569989875, -0.04519816824222727, 0.02302416556181928,
    -0.007568498443339108, 0.0011681262903039157,
)


def _prep_body(xe_ref, xh_ref, xs_ref, w_ref, hist_ref, hs_ref, dinv_ref):
    xe = xe_ref[...]
    xh = xh_ref[...]
    xs = xs_ref[...]

    nh = jnp.sqrt(jnp.sum(xh * xh, axis=1, keepdims=True))
    nhc = jnp.clip(nh, 1e-8, 1.0 - 1e-5)
    ath = 0.5 * jnp.log((1.0 + nhc) / (1.0 - nhc))
    xh = ath * xh / jnp.maximum(nh, 1e-8)

    nsr = jnp.sqrt(jnp.sum(xs * xs, axis=1, keepdims=True))
    nsc = jnp.maximum(nsr, 1e-8)
    t = jnp.minimum(nsc, 1.0 / nsc)
    u = t * t
    p = _ATAN_COEF[-1]
    for coef in _ATAN_COEF[-2::-1]:
        p = p * u + coef
    p = p * t
    atn = jnp.where(nsc <= 1.0, p, (jnp.pi / 2) - p)
    xs = atn * xs / nsc

    w = w_ref[...]
    h = jnp.dot(xe, w[0:D], preferred_element_type=jnp.float32)
    h += jnp.dot(xh, w[D:2 * D], preferred_element_type=jnp.float32)
    h += jnp.dot(xs, w[2 * D:3 * D], preferred_element_type=jnp.float32)

    deg = jnp.sum(hist_ref[0], axis=0, keepdims=True) + 1.0
    dinv = lax.rsqrt(deg)
    hs_ref[...] = h * dinv.T
    dinv_ref[...] = dinv.T


_PREP_R = 1280


def _tc_prep(x_E, x_H, x_S, W, hist):
    return pl.pallas_call(
        _prep_body,
        grid=(pl.cdiv(N, _PREP_R),),
        in_specs=[
            pl.BlockSpec((_PREP_R, D), lambda i: (i, 0)),
            pl.BlockSpec((_PREP_R, D), lambda i: (i, 0)),
            pl.BlockSpec((_PREP_R, D), lambda i: (i, 0)),
            pl.BlockSpec((3 * D, CLS_DIM), lambda i: (0, 0)),
            pl.BlockSpec((1, NW, _HIST_G), lambda i: (i, 0, 0)),
        ],
        out_specs=[
            pl.BlockSpec((_PREP_R, CLS_DIM), lambda i: (i, 0)),
            pl.BlockSpec((_PREP_R, 1), lambda i: (i, 0)),
        ],
        out_shape=[
            jax.ShapeDtypeStruct((NPAD, CLS_DIM), jnp.float32),
            jax.ShapeDtypeStruct((N, 1), jnp.float32),
        ],
    )(x_E, x_H, x_S, W, hist)


CHUNK = 80
BIGC = 2000


@functools.partial(
    pl.kernel,
    out_type=jax.ShapeDtypeStruct((NC, NPAD, CLS_DIM), jnp.float32),
    mesh=_sc_mesh,
    scratch_types=[
        pltpu.VMEM_SHARED((NPAD, CLS_DIM), jnp.float32),
        pltpu.VMEM((BIGC,), jnp.int32),
        pltpu.VMEM((BIGC,), jnp.int32),
        pltpu.VMEM((CHUNK,), jnp.int32),
        pltpu.VMEM((3, CHUNK, CLS_DIM), jnp.float32),
        pltpu.SemaphoreType.DMA,
        pltpu.SemaphoreType.DMA,
    ],
)
def _scatter_kernel(hs_hbm, ei_hbm, s_out,
                    acc, srcbig, dstbig, dst_v, rows_v, gsem, ssem):
    c = lax.axis_index("c")
    s = lax.axis_index("s")
    wid = c * NS + s
    rbase = s * RPTP

    pltpu.sync_copy(hs_hbm.at[pl.ds(rbase, RPTP)], acc.at[pl.ds(rbase, RPTP)])
    plsc.subcore_barrier()

    ebase = wid * EPW
    NCH = BIGC // CHUNK

    def _gdrain(slot):
        pltpu.make_async_copy(hs_hbm.at[pl.ds(0, CHUNK)],
                              rows_v.at[slot], gsem).wait()

    def _sdrain(slot):
        pltpu.make_async_copy(rows_v.at[slot],
                              acc.at[pl.ds(0, CHUNK)], ssem).wait()

    def _gath(k, slot):
        pltpu.async_copy(hs_hbm.at[srcbig.at[pl.ds(k * CHUNK, CHUNK)]],
                         rows_v.at[slot], gsem)

    def _scat(k, slot):
        for v in range(CHUNK // 16):
            dst_v[pl.ds(v * 16, 16)] = dstbig[pl.ds(k * CHUNK + v * 16, 16)]
        pltpu.async_copy(rows_v.at[slot], acc.at[dst_v], ssem, add=True)

    def big_body(bi, _):
        bb = ebase + bi * BIGC
        pltpu.sync_copy(ei_hbm.at[pl.ds(bb, BIGC)], srcbig)
        pltpu.sync_copy(ei_hbm.at[pl.ds(E + bb, BIGC)], dstbig)
        _gath(0, 0)
        _gath(1, 1)

        def tri(t, _):
            for off in range(3):
                j = 3 * t + off
                sp = (off - 1) % 3
                sn = (off + 2) % 3

                @pl.when(j >= 1)
                def _():
                    _sdrain(sp)

                @pl.when(j + 2 < NCH)
                def _():
                    _gath(j + 2, sn)

                _gdrain(off)
                _scat(j, off)
            return 0

        lax.fori_loop(0, NCH // 3, tri, 0)
        _sdrain(2)
        _gdrain(0)
        _scat(NCH - 1, 0)
        _sdrain(0)
        return 0

    lax.fori_loop(0, EPW // BIGC, big_body, 0)
    plsc.subcore_barrier()
    pltpu.sync_copy(acc.at[pl.ds(rbase, RPTP)], s_out.at[c, pl.ds(rbase, RPTP)])


def _out_body(s_ref, hs_ref, dinv_ref, b_ref, cls_ref, out_ref):
    y = (s_ref[0] + s_ref[1] - hs_ref[...]) * dinv_ref[...] + b_ref[...]
    yn = jnp.sqrt(jnp.sum(y * y, axis=1, keepdims=True))
    y = y / jnp.maximum(yn, 1e-8)
    cemb = cls_ref[...]
    cn = jnp.sqrt(jnp.sum(cemb * cemb, axis=1, keepdims=True))
    cemb = cemb / jnp.maximum(cn, 1e-8)
    out_ref[...] = lax.dot_general(
        y, cemb, (((1,), (1,)), ((), ())),
        preferred_element_type=jnp.float32)


_OUT_R = 1000


def _tc_out(s_part, hs, dinv, b2, cls_embeddings):
    return pl.pallas_call(
        _out_body,
        grid=(N // _OUT_R,),
        in_specs=[
            pl.BlockSpec((NC, _OUT_R, CLS_DIM), lambda i: (0, i, 0)),
            pl.BlockSpec((_OUT_R, CLS_DIM), lambda i: (i, 0)),
            pl.BlockSpec((_OUT_R, 1), lambda i: (i, 0)),
            pl.BlockSpec((1, CLS_DIM), lambda i: (0, 0)),
            pl.BlockSpec((NUM_CLS, CLS_DIM), lambda i: (0, 0)),
        ],
        out_specs=pl.BlockSpec((_OUT_R, NUM_CLS), lambda i: (i, 0)),
        out_shape=jax.ShapeDtypeStruct((N, NUM_CLS), jnp.float32),
    )(s_part, hs, dinv, b2, cls_embeddings)


def kernel(x_E, x_H, x_S, edge_index, W, b, cls_embeddings):
    ei = edge_index.reshape(2 * E)
    hist = _hist_kernel(ei)
    hs, dinv = _tc_prep(x_E, x_H, x_S, W, hist)
    s_part = _scatter_kernel(hs, ei)
    return _tc_out(s_part, hs, dinv, b.reshape(1, CLS_DIM), cls_embeddings)
